# baseline (speedup 1.0000x reference)
import functools

import jax
import jax.numpy as jnp
import numpy as np
from jax import lax
from jax.experimental import pallas as pl
from jax.experimental.pallas import tpu as pltpu
from jax.experimental.pallas import tpu_sc as plsc

L_SEQ = 1024
DIM = 128
NUM_NEAREST = 100
NUM_TIMES = 169
K_NEG = 20
CS = 512
R_ROWS = 128
NCP = 256
GROUP = 4
INT_MAX = int(np.iinfo(np.int32).max)
BIG = 2**30
PAD_D2 = float(np.float32(3e38))


_SC_WORKERS = 32


def _gather_q(table, idx):
    b = idx.shape[0]
    d = table.shape[1]
    bpw = b // _SC_WORKERS
    mesh = plsc.VectorSubcoreMesh(core_axis_name="c", subcore_axis_name="s")

    @functools.partial(
        pl.kernel,
        mesh=mesh,
        out_type=jax.ShapeDtypeStruct((b, d), jnp.float32),
        scratch_types=[
            pltpu.VMEM((bpw,), jnp.int32),
            pltpu.VMEM((bpw, d), jnp.float32),
            pltpu.SemaphoreType.DMA,
        ],
    )
    def gk(table_hbm, idx_hbm, out_hbm, idx_v, rows_v, sem):
        wid = lax.axis_index("s") * 2 + lax.axis_index("c")
        base = wid * bpw
        pltpu.sync_copy(idx_hbm.at[pl.ds(base, bpw)], idx_v)
        pltpu.async_copy(table_hbm.at[idx_v], rows_v, sem).wait()
        pltpu.sync_copy(rows_v, out_hbm.at[pl.ds(base, bpw)])

    return gk(table, idx)


def _f32_key(x):
    b = lax.bitcast_convert_type(x, jnp.int32)
    return b ^ ((b >> 31) & jnp.int32(0x7FFFFFFF))


def _sq_prepass(loc_embeds, nc, v):
    sq = jnp.sum(loc_embeds * loc_embeds, axis=1)
    sq = jnp.pad(sq, (0, nc * CS - v), constant_values=np.float32(PAD_D2))
    return sq.reshape(nc, 1, CS)


def _main_body(q_ref, e_ref, sq_ref, sel_ref, rr_ref, tid_ref,
               neg_ref, tneg_ref, u_s, m2_s, ms_s, knn_s,
               *, nc, ncp, r_rows, cs, k_sel, n_near, group):
    c = pl.program_id(1)

    @pl.when(c == 0)
    def _():
        m2_s[...] = jnp.full(m2_s.shape, INT_MAX, jnp.int32)

    scores = lax.dot_general(q_ref[...], e_ref[...], (((1,), (1,)), ((), ())),
                             preferred_element_type=jnp.float32)
    d2 = sq_ref[0] - 2.0 * scores
    key = _f32_key(d2)
    u_s[c] = key
    colmin = jnp.min(key, axis=1)
    m2_s[c] = colmin.reshape(1, r_rows)

    @pl.when(c == nc - 1)
    def _():
        ms_s[...] = m2_s[:, 0, :].T

        iota_c = lax.broadcasted_iota(jnp.int32, (1, ncp), 1)
        iota_l = lax.broadcasted_iota(jnp.int32, (1, cs), 1)
        iota_k = lax.broadcasted_iota(jnp.int32, (1, 128), 1)

        def row_group(i0, _):
            rows = [i0 * group + g for g in range(group)]
            m_rows = tuple(ms_s[pl.ds(row, 1), :] for row in rows)
            out_rows = tuple(jnp.zeros((1, 128), jnp.int32) for _ in rows)

            def step_fn(s, carry):
                m_rows, out_rows = carry
                new_m, new_o = [], []
                for g in range(group):
                    row = rows[g]
                    m_row = m_rows[g]
                    m = jnp.min(m_row)
                    cstar = jnp.min(jnp.where(m_row == m, iota_c, BIG))
                    chunk = u_s[cstar, pl.ds(row, 1), :]
                    local = jnp.min(jnp.where(chunk == m, iota_l, BIG))
                    gidx = cstar * cs + local
                    new_o.append(jnp.where(iota_k == s, gidx, out_rows[g]))
                    chunk2 = jnp.where(iota_l == local, INT_MAX, chunk)
                    u_s[cstar, pl.ds(row, 1), :] = chunk2
                    new_m.append(jnp.where(iota_c == cstar, jnp.min(chunk2),
                                           m_row))
                return (tuple(new_m), tuple(new_o))

            m_rows, out_rows = lax.fori_loop(0, n_near, step_fn,
                                             (m_rows, out_rows))
            for g in range(group):
                knn_s[pl.ds(rows[g], 1), :] = out_rows[g]
            return 0

        lax.fori_loop(0, r_rows // group, row_group, 0)

        knn = knn_s[...]
        for j in range(k_sel):
            maskj = sel_ref[:, pl.ds(j, 1)] == iota_k
            negj = jnp.sum(jnp.where(maskj, knn, 0), axis=1, keepdims=True)
            neg_ref[:, pl.ds(j, 1)] = negj
        rr = rr_ref[...]
        tneg_ref[...] = rr + (rr >= tid_ref[...]).astype(jnp.int32)


def _main(q, e_pad, sq, sel, rr, tid2d, nc, r_rows):
    l_seq, d = q.shape
    k_sel = sel.shape[1]
    rt = l_seq // r_rows
    body = functools.partial(
        _main_body, nc=nc, ncp=NCP, r_rows=r_rows, cs=CS,
        k_sel=k_sel, n_near=NUM_NEAREST, group=GROUP)
    return pl.pallas_call(
        body,
        grid=(rt, nc),
        in_specs=[
            pl.BlockSpec((r_rows, d), lambda rt, c: (rt, 0)),
            pl.BlockSpec((CS, d), lambda rt, c: (c, 0)),
            pl.BlockSpec((1, 1, CS), lambda rt, c: (c, 0, 0)),
            pl.BlockSpec((r_rows, k_sel), lambda rt, c: (rt, 0)),
            pl.BlockSpec((r_rows, k_sel), lambda rt, c: (rt, 0)),
            pl.BlockSpec((r_rows, 1), lambda rt, c: (rt, 0)),
        ],
        out_specs=[
            pl.BlockSpec((r_rows, k_sel), lambda rt, c: (rt, 0)),
            pl.BlockSpec((r_rows, k_sel), lambda rt, c: (rt, 0)),
        ],
        out_shape=[
            jax.ShapeDtypeStruct((l_seq, k_sel), jnp.int32),
            jax.ShapeDtypeStruct((l_seq, k_sel), jnp.int32),
        ],
        scratch_shapes=[
            pltpu.VMEM((nc, r_rows, CS), jnp.int32),
            pltpu.VMEM((NCP, 1, r_rows), jnp.int32),
            pltpu.VMEM((r_rows, NCP), jnp.int32),
            pltpu.VMEM((r_rows, 128), jnp.int32),
        ],
        compiler_params=pltpu.CompilerParams(
            dimension_semantics=("arbitrary", "arbitrary"),
            vmem_limit_bytes=100 * 1024 * 1024,
        ),
    )(q, e_pad, sq, sel, rr, tid2d)


def kernel(POI_id, time_id, loc_embeds, k, user):
    key = jax.random.key(42)
    ks, kt = jax.random.split(key)
    sel = jax.random.randint(ks, (L_SEQ, K_NEG), 0, NUM_NEAREST,
                             dtype=jnp.int32)
    rr = jax.random.randint(kt, (L_SEQ, K_NEG), 0, NUM_TIMES - 1,
                            dtype=jnp.int32)

    v = loc_embeds.shape[0]
    nc = -(-v // CS)
    e_pad = jnp.pad(loc_embeds, ((0, nc * CS - v), (0, 0)))
    q = _gather_q(loc_embeds, POI_id)
    sq = _sq_prepass(loc_embeds, nc, v)
    tid2d = time_id.reshape(-1, 1).astype(jnp.int32)
    neg, tneg = _main(q, e_pad, sq, sel, rr, tid2d, nc, R_ROWS)
    probs = jnp.ones((L_SEQ, K_NEG), jnp.float32)
    return (neg, probs, tneg, jnp.ones((L_SEQ, K_NEG), jnp.float32))


# speedup vs baseline: 3.6482x; 3.6482x over previous
import functools

import jax
import jax.numpy as jnp
import numpy as np
from jax import lax
from jax.experimental import pallas as pl
from jax.experimental.pallas import tpu as pltpu
from jax.experimental.pallas import tpu_sc as plsc

L_SEQ = 1024
DIM = 128
NUM_NEAREST = 100
NUM_TIMES = 169
K_NEG = 20
CS = 512
R_ROWS = 128
NCP = 256
GROUP = 4
INT_MAX = int(np.iinfo(np.int32).max)
PAD_D2 = float(np.float32(3e38))
FMAX = float(np.finfo(np.float32).max)


_SC_WORKERS = 32


def _gather_q(table, idx):
    b = idx.shape[0]
    d = table.shape[1]
    bpw = b // _SC_WORKERS
    mesh = plsc.VectorSubcoreMesh(core_axis_name="c", subcore_axis_name="s")

    @functools.partial(
        pl.kernel,
        mesh=mesh,
        out_type=jax.ShapeDtypeStruct((b, d), jnp.float32),
        scratch_types=[
            pltpu.VMEM((bpw,), jnp.int32),
            pltpu.VMEM((bpw, d), jnp.float32),
            pltpu.SemaphoreType.DMA,
        ],
    )
    def gk(table_hbm, idx_hbm, out_hbm, idx_v, rows_v, sem):
        wid = lax.axis_index("s") * 2 + lax.axis_index("c")
        base = wid * bpw
        pltpu.sync_copy(idx_hbm.at[pl.ds(base, bpw)], idx_v)
        pltpu.async_copy(table_hbm.at[idx_v], rows_v, sem).wait()
        pltpu.sync_copy(rows_v, out_hbm.at[pl.ds(base, bpw)])

    return gk(table, idx)


def _f32_key(x):
    b = lax.bitcast_convert_type(x, jnp.int32)
    return b ^ ((b >> 31) & jnp.int32(0x7FFFFFFF))


def _sq_prepass(loc_embeds, nc, v):
    sq = jnp.sum(loc_embeds * loc_embeds, axis=1)
    sq = jnp.pad(sq, (0, nc * CS - v), constant_values=np.float32(PAD_D2))
    return sq.reshape(nc, 1, CS)


def _ce(k_arr, i_arr, j, dir_mask, iota):
    upper = (iota & j) != 0
    pk = jnp.where(upper, jnp.roll(k_arr, j, axis=1), jnp.roll(k_arr, -j, axis=1))
    pi = jnp.where(upper, jnp.roll(i_arr, j, axis=1), jnp.roll(i_arr, -j, axis=1))
    psm = (pk < k_arr) | ((pk == k_arr) & (pi < i_arr))
    take = psm ^ upper ^ dir_mask
    return jnp.where(take, pk, k_arr), jnp.where(take, pi, i_arr)


def _main_body(q_ref, e_ref, sq_ref, sel_ref, rr_ref, tid_ref,
               neg_ref, tneg_ref, ak_s, ai_s,
               *, nc, r_rows, cs, k_sel, n_keep):
    c = pl.program_id(1)
    iota_cs = lax.broadcasted_iota(jnp.int32, (1, cs), 1)
    iota_k = lax.broadcasted_iota(jnp.int32, (1, n_keep), 1)

    @pl.when(c == 0)
    def _():
        ak_s[...] = jnp.full(ak_s.shape, jnp.float32(FMAX), jnp.float32)
        ai_s[...] = jnp.zeros(ai_s.shape, jnp.int32)

    scores = lax.dot_general(q_ref[...], e_ref[...], (((1,), (1,)), ((), ())),
                             preferred_element_type=jnp.float32)
    d2 = sq_ref[0] - 2.0 * scores

    nsub = cs // n_keep
    subs = []
    for sub in range(nsub):
        k_arr = d2[:, sub * n_keep:(sub + 1) * n_keep]
        i_arr = jnp.broadcast_to(c * cs + sub * n_keep + iota_k, k_arr.shape)
        subs.append((k_arr, i_arr))
    kk = 2
    while kk <= n_keep:
        j = kk // 2
        while j >= 1:
            dir_asc = (iota_k & kk) != 0
            dir_desc = ~dir_asc
            subs = [_ce(*subs[s], j, dir_asc if s % 2 == 0 else dir_desc,
                        iota_k) for s in range(nsub)]
            j //= 2
        kk *= 2

    def lexmin(a, b):
        cond = (a[0] < b[0]) | ((a[0] == b[0]) & (a[1] < b[1]))
        return jnp.where(cond, a[0], b[0]), jnp.where(cond, a[1], b[1])

    def bmerge(pair, descending):
        d = jnp.full((1, n_keep), descending, bool)
        j = n_keep // 2
        while j >= 1:
            pair = _ce(*pair, j, d, iota_k)
            j //= 2
        return pair

    t1 = bmerge(lexmin(subs[0], subs[1]), False)
    t2 = bmerge(lexmin(subs[2], subs[3]), True)
    u = bmerge(lexmin(t1, t2), True)
    a_new = bmerge(lexmin((ak_s[...], ai_s[...]), u), False)
    ak_s[...] = a_new[0]
    ai_s[...] = a_new[1]

    @pl.when(c == nc - 1)
    def _():
        knn = ai_s[...]
        for j in range(k_sel):
            maskj = sel_ref[:, pl.ds(j, 1)] == iota_k
            negj = jnp.sum(jnp.where(maskj, knn, 0), axis=1, keepdims=True)
            neg_ref[:, pl.ds(j, 1)] = negj
        rr = rr_ref[...]
        tneg_ref[...] = rr + (rr >= tid_ref[...]).astype(jnp.int32)


def _main(q, e_pad, sq, sel, rr, tid2d, nc, r_rows):
    l_seq, d = q.shape
    k_sel = sel.shape[1]
    rt = l_seq // r_rows
    body = functools.partial(
        _main_body, nc=nc, r_rows=r_rows, cs=CS, k_sel=k_sel, n_keep=128)
    return pl.pallas_call(
        body,
        grid=(rt, nc),
        in_specs=[
            pl.BlockSpec((r_rows, d), lambda rt, c: (rt, 0)),
            pl.BlockSpec((CS, d), lambda rt, c: (c, 0)),
            pl.BlockSpec((1, 1, CS), lambda rt, c: (c, 0, 0)),
            pl.BlockSpec((r_rows, k_sel), lambda rt, c: (rt, 0)),
            pl.BlockSpec((r_rows, k_sel), lambda rt, c: (rt, 0)),
            pl.BlockSpec((r_rows, 1), lambda rt, c: (rt, 0)),
        ],
        out_specs=[
            pl.BlockSpec((r_rows, k_sel), lambda rt, c: (rt, 0)),
            pl.BlockSpec((r_rows, k_sel), lambda rt, c: (rt, 0)),
        ],
        out_shape=[
            jax.ShapeDtypeStruct((l_seq, k_sel), jnp.int32),
            jax.ShapeDtypeStruct((l_seq, k_sel), jnp.int32),
        ],
        scratch_shapes=[
            pltpu.VMEM((r_rows, 128), jnp.float32),
            pltpu.VMEM((r_rows, 128), jnp.int32),
        ],
        compiler_params=pltpu.CompilerParams(
            dimension_semantics=("arbitrary", "arbitrary"),
            vmem_limit_bytes=100 * 1024 * 1024,
        ),
    )(q, e_pad, sq, sel, rr, tid2d)


def kernel(POI_id, time_id, loc_embeds, k, user):
    key = jax.random.key(42)
    ks, kt = jax.random.split(key)
    sel = jax.random.randint(ks, (L_SEQ, K_NEG), 0, NUM_NEAREST,
                             dtype=jnp.int32)
    rr = jax.random.randint(kt, (L_SEQ, K_NEG), 0, NUM_TIMES - 1,
                            dtype=jnp.int32)

    v = loc_embeds.shape[0]
    nc = -(-v // CS)
    e_pad = jnp.pad(loc_embeds, ((0, nc * CS - v), (0, 0)))
    q = _gather_q(loc_embeds, POI_id)
    sq = _sq_prepass(loc_embeds, nc, v)
    tid2d = time_id.reshape(-1, 1).astype(jnp.int32)
    neg, tneg = _main(q, e_pad, sq, sel, rr, tid2d, nc, R_ROWS)
    probs = jnp.ones((L_SEQ, K_NEG), jnp.float32)
    return (neg, probs, tneg, jnp.ones((L_SEQ, K_NEG), jnp.float32))


# speedup vs baseline: 4.2339x; 1.1605x over previous
import functools

import jax
import jax.numpy as jnp
import numpy as np
from jax import lax
from jax.experimental import pallas as pl
from jax.experimental.pallas import tpu as pltpu
from jax.experimental.pallas import tpu_sc as plsc

L_SEQ = 1024
DIM = 128
NUM_NEAREST = 100
NUM_TIMES = 169
K_NEG = 20
CS = 512
R_ROWS = 128
NCP = 256
GROUP = 4
INT_MAX = int(np.iinfo(np.int32).max)
PAD_D2 = float(np.float32(3e38))
FMAX = float(np.finfo(np.float32).max)


_SC_WORKERS = 32


def _gather_q(table, idx):
    b = idx.shape[0]
    d = table.shape[1]
    bpw = b // _SC_WORKERS
    mesh = plsc.VectorSubcoreMesh(core_axis_name="c", subcore_axis_name="s")

    @functools.partial(
        pl.kernel,
        mesh=mesh,
        out_type=jax.ShapeDtypeStruct((b, d), jnp.float32),
        scratch_types=[
            pltpu.VMEM((bpw,), jnp.int32),
            pltpu.VMEM((bpw, d), jnp.float32),
            pltpu.SemaphoreType.DMA,
        ],
    )
    def gk(table_hbm, idx_hbm, out_hbm, idx_v, rows_v, sem):
        wid = lax.axis_index("s") * 2 + lax.axis_index("c")
        base = wid * bpw
        pltpu.sync_copy(idx_hbm.at[pl.ds(base, bpw)], idx_v)
        pltpu.async_copy(table_hbm.at[idx_v], rows_v, sem).wait()
        pltpu.sync_copy(rows_v, out_hbm.at[pl.ds(base, bpw)])

    return gk(table, idx)


def _f32_key(x):
    b = lax.bitcast_convert_type(x, jnp.int32)
    return b ^ ((b >> 31) & jnp.int32(0x7FFFFFFF))


def _sq_prepass(loc_embeds, nc, v):
    sq = jnp.sum(loc_embeds * loc_embeds, axis=1)
    sq = jnp.pad(sq, (0, nc * CS - v), constant_values=np.float32(PAD_D2))
    return sq.reshape(nc, 1, CS)


def _ce(k_arr, i_arr, j, dir_mask, iota):
    upper = (iota & j) != 0
    pk = jnp.where(upper, jnp.roll(k_arr, j, axis=1), jnp.roll(k_arr, -j, axis=1))
    pi = jnp.where(upper, jnp.roll(i_arr, j, axis=1), jnp.roll(i_arr, -j, axis=1))
    psm = (pk < k_arr) | ((pk == k_arr) & (pi < i_arr))
    take = psm ^ upper ^ dir_mask
    return jnp.where(take, pk, k_arr), jnp.where(take, pi, i_arr)


def _main_body(q_ref, e_ref, sq_ref, sel_ref, rr_ref, tid_ref,
               neg_ref, tneg_ref, ak_s, ai_s,
               *, nc, r_rows, cs, k_sel, n_keep):
    c = pl.program_id(1)
    iota_cs = lax.broadcasted_iota(jnp.int32, (1, cs), 1)
    iota_k = lax.broadcasted_iota(jnp.int32, (1, n_keep), 1)

    nsub = cs // n_keep

    @pl.when(c == 0)
    def _():
        ak_s[...] = jnp.full(ak_s.shape, jnp.float32(FMAX), jnp.float32)
        ai_s[...] = jnp.zeros(ai_s.shape, jnp.int32)

    def lexmin(a, b):
        cond = (a[0] < b[0]) | ((a[0] == b[0]) & (a[1] < b[1]))
        return jnp.where(cond, a[0], b[0]), jnp.where(cond, a[1], b[1])

    def bmerge(pair, descending):
        d = jnp.full((1, n_keep), descending, bool)
        j = n_keep // 2
        while j >= 1:
            pair = _ce(*pair, j, d, iota_k)
            j //= 2
        return pair

    scores = lax.dot_general(q_ref[...], e_ref[...], (((1,), (1,)), ((), ())),
                             preferred_element_type=jnp.float32)
    d2 = sq_ref[0] - 2.0 * scores

    subs = []
    for sub in range(nsub):
        k_arr = d2[:, sub * n_keep:(sub + 1) * n_keep]
        i_arr = jnp.broadcast_to(c * cs + sub * n_keep + iota_k, k_arr.shape)
        subs.append((k_arr, i_arr))
    kk = 2
    while kk <= n_keep:
        j = kk // 2
        while j >= 1:
            dir_desc = (iota_k & kk) == 0
            subs = [_ce(*subs[s], j, dir_desc, iota_k) for s in range(nsub)]
            j //= 2
        kk *= 2

    pairs = [lexmin((ak_s[s], ai_s[s]), subs[s]) for s in range(nsub)]
    j = n_keep // 2
    zero_dir = jnp.zeros((1, n_keep), bool)
    while j >= 1:
        pairs = [_ce(*pairs[s], j, zero_dir, iota_k) for s in range(nsub)]
        j //= 2
    for s in range(nsub):
        ak_s[s] = pairs[s][0]
        ai_s[s] = pairs[s][1]

    @pl.when(c == nc - 1)
    def _():
        a0 = (ak_s[0], ai_s[0])
        a1d = bmerge((ak_s[1], ai_s[1]), True)
        a2 = (ak_s[2], ai_s[2])
        a3d = bmerge((ak_s[3], ai_s[3]), True)
        t1 = bmerge(lexmin(a0, a1d), False)
        t2 = bmerge(lexmin(a2, a3d), True)
        knn = bmerge(lexmin(t1, t2), False)[1]

        for j in range(k_sel):
            maskj = sel_ref[:, pl.ds(j, 1)] == iota_k
            negj = jnp.sum(jnp.where(maskj, knn, 0), axis=1, keepdims=True)
            neg_ref[:, pl.ds(j, 1)] = negj
        rr = rr_ref[...]
        tneg_ref[...] = rr + (rr >= tid_ref[...]).astype(jnp.int32)


def _main(q, e_pad, sq, sel, rr, tid2d, nc, r_rows):
    l_seq, d = q.shape
    k_sel = sel.shape[1]
    rt = l_seq // r_rows
    body = functools.partial(
        _main_body, nc=nc, r_rows=r_rows, cs=CS, k_sel=k_sel, n_keep=128)
    return pl.pallas_call(
        body,
        grid=(rt, nc),
        in_specs=[
            pl.BlockSpec((r_rows, d), lambda rt, c: (rt, 0)),
            pl.BlockSpec((CS, d), lambda rt, c: (c, 0)),
            pl.BlockSpec((1, 1, CS), lambda rt, c: (c, 0, 0)),
            pl.BlockSpec((r_rows, k_sel), lambda rt, c: (rt, 0)),
            pl.BlockSpec((r_rows, k_sel), lambda rt, c: (rt, 0)),
            pl.BlockSpec((r_rows, 1), lambda rt, c: (rt, 0)),
        ],
        out_specs=[
            pl.BlockSpec((r_rows, k_sel), lambda rt, c: (rt, 0)),
            pl.BlockSpec((r_rows, k_sel), lambda rt, c: (rt, 0)),
        ],
        out_shape=[
            jax.ShapeDtypeStruct((l_seq, k_sel), jnp.int32),
            jax.ShapeDtypeStruct((l_seq, k_sel), jnp.int32),
        ],
        scratch_shapes=[
            pltpu.VMEM((CS // 128, r_rows, 128), jnp.float32),
            pltpu.VMEM((CS // 128, r_rows, 128), jnp.int32),
        ],
        compiler_params=pltpu.CompilerParams(
            dimension_semantics=("arbitrary", "arbitrary"),
            vmem_limit_bytes=100 * 1024 * 1024,
        ),
    )(q, e_pad, sq, sel, rr, tid2d)


def kernel(POI_id, time_id, loc_embeds, k, user):
    key = jax.random.key(42)
    ks, kt = jax.random.split(key)
    sel = jax.random.randint(ks, (L_SEQ, K_NEG), 0, NUM_NEAREST,
                             dtype=jnp.int32)
    rr = jax.random.randint(kt, (L_SEQ, K_NEG), 0, NUM_TIMES - 1,
                            dtype=jnp.int32)

    v = loc_embeds.shape[0]
    nc = -(-v // CS)
    e_pad = jnp.pad(loc_embeds, ((0, nc * CS - v), (0, 0)))
    q = _gather_q(loc_embeds, POI_id)
    sq = _sq_prepass(loc_embeds, nc, v)
    tid2d = time_id.reshape(-1, 1).astype(jnp.int32)
    neg, tneg = _main(q, e_pad, sq, sel, rr, tid2d, nc, R_ROWS)
    probs = jnp.ones((L_SEQ, K_NEG), jnp.float32)
    return (neg, probs, tneg, jnp.ones((L_SEQ, K_NEG), jnp.float32))


# speedup vs baseline: 4.2351x; 1.0003x over previous
import functools

import jax
import jax.numpy as jnp
import numpy as np
from jax import lax
from jax.experimental import pallas as pl
from jax.experimental.pallas import tpu as pltpu
from jax.experimental.pallas import tpu_sc as plsc

L_SEQ = 1024
DIM = 128
NUM_NEAREST = 100
NUM_TIMES = 169
K_NEG = 20
CS = 512
R_ROWS = 128
PAD_D2 = float(np.float32(3e38))
FMAX = float(np.finfo(np.float32).max)


_SC_WORKERS = 32


def _gather_q(table, idx):
    b = idx.shape[0]
    d = table.shape[1]
    bpw = b // _SC_WORKERS
    mesh = plsc.VectorSubcoreMesh(core_axis_name="c", subcore_axis_name="s")

    @functools.partial(
        pl.kernel,
        mesh=mesh,
        out_type=jax.ShapeDtypeStruct((b, d), jnp.float32),
        scratch_types=[
            pltpu.VMEM((bpw,), jnp.int32),
            pltpu.VMEM((bpw, d), jnp.float32),
            pltpu.SemaphoreType.DMA,
        ],
    )
    def gk(table_hbm, idx_hbm, out_hbm, idx_v, rows_v, sem):
        wid = lax.axis_index("s") * 2 + lax.axis_index("c")
        base = wid * bpw
        pltpu.sync_copy(idx_hbm.at[pl.ds(base, bpw)], idx_v)
        pltpu.async_copy(table_hbm.at[idx_v], rows_v, sem).wait()
        pltpu.sync_copy(rows_v, out_hbm.at[pl.ds(base, bpw)])

    return gk(table, idx)


def _sq_prepass(loc_embeds, nc, v):
    sq = jnp.sum(loc_embeds * loc_embeds, axis=1)
    sq = jnp.pad(sq, (0, nc * CS - v), constant_values=np.float32(PAD_D2))
    return sq.reshape(nc, 1, CS)


def _ce(k_arr, i_arr, j, dir_mask, iota):
    upper = (iota & j) != 0
    pk = jnp.where(upper, jnp.roll(k_arr, j, axis=1), jnp.roll(k_arr, -j, axis=1))
    pi = jnp.where(upper, jnp.roll(i_arr, j, axis=1), jnp.roll(i_arr, -j, axis=1))
    psm = (pk < k_arr) | ((pk == k_arr) & (pi < i_arr))
    take = psm ^ upper ^ dir_mask
    return jnp.where(take, pk, k_arr), jnp.where(take, pi, i_arr)


def _main_body(q_ref, e_ref, sq_ref, sel_ref, rr_ref, tid_ref,
               neg_ref, tneg_ref, ak_s, ai_s,
               *, nc, r_rows, cs, k_sel, n_keep):
    c = pl.program_id(1)
    iota_k = lax.broadcasted_iota(jnp.int32, (1, n_keep), 1)

    nsub = cs // n_keep

    @pl.when(c == 0)
    def _():
        ak_s[...] = jnp.full(ak_s.shape, jnp.float32(FMAX), jnp.float32)
        ai_s[...] = jnp.zeros(ai_s.shape, jnp.int32)

    def lexmin(a, b):
        cond = (a[0] < b[0]) | ((a[0] == b[0]) & (a[1] < b[1]))
        return jnp.where(cond, a[0], b[0]), jnp.where(cond, a[1], b[1])

    def bmerge(pair, descending):
        d = jnp.full((1, n_keep), descending, bool)
        j = n_keep // 2
        while j >= 1:
            pair = _ce(*pair, j, d, iota_k)
            j //= 2
        return pair

    scores = lax.dot_general(q_ref[...], e_ref[...], (((1,), (1,)), ((), ())),
                             preferred_element_type=jnp.float32)
    d2 = sq_ref[0] - 2.0 * scores

    subs = []
    for sub in range(nsub):
        k_arr = d2[:, sub * n_keep:(sub + 1) * n_keep]
        i_arr = jnp.broadcast_to(c * cs + sub * n_keep + iota_k, k_arr.shape)
        subs.append((k_arr, i_arr))
    kk = 2
    while kk <= n_keep:
        j = kk // 2
        while j >= 1:
            dir_desc = (iota_k & kk) == 0
            subs = [_ce(*subs[s], j, dir_desc, iota_k) for s in range(nsub)]
            j //= 2
        kk *= 2

    pairs = [lexmin((ak_s[s], ai_s[s]), subs[s]) for s in range(nsub)]
    j = n_keep // 2
    zero_dir = jnp.zeros((1, n_keep), bool)
    while j >= 1:
        pairs = [_ce(*pairs[s], j, zero_dir, iota_k) for s in range(nsub)]
        j //= 2
    for s in range(nsub):
        ak_s[s] = pairs[s][0]
        ai_s[s] = pairs[s][1]

    @pl.when(c == nc - 1)
    def _():
        a0 = (ak_s[0], ai_s[0])
        a1d = bmerge((ak_s[1], ai_s[1]), True)
        a2 = (ak_s[2], ai_s[2])
        a3d = bmerge((ak_s[3], ai_s[3]), True)
        t1 = bmerge(lexmin(a0, a1d), False)
        t2 = bmerge(lexmin(a2, a3d), True)
        knn = bmerge(lexmin(t1, t2), False)[1]

        for j in range(k_sel):
            maskj = sel_ref[:, pl.ds(j, 1)] == iota_k
            negj = jnp.sum(jnp.where(maskj, knn, 0), axis=1, keepdims=True)
            neg_ref[:, pl.ds(j, 1)] = negj
        rr = rr_ref[...]
        tneg_ref[...] = rr + (rr >= tid_ref[...]).astype(jnp.int32)


def _main(q, e_pad, sq, sel, rr, tid2d, nc, r_rows):
    l_seq, d = q.shape
    k_sel = sel.shape[1]
    rt = l_seq // r_rows
    body = functools.partial(
        _main_body, nc=nc, r_rows=r_rows, cs=CS, k_sel=k_sel, n_keep=128)
    return pl.pallas_call(
        body,
        grid=(rt, nc),
        in_specs=[
            pl.BlockSpec((r_rows, d), lambda rt, c: (rt, 0)),
            pl.BlockSpec((CS, d), lambda rt, c: (c, 0)),
            pl.BlockSpec((1, 1, CS), lambda rt, c: (c, 0, 0)),
            pl.BlockSpec((r_rows, k_sel), lambda rt, c: (rt, 0)),
            pl.BlockSpec((r_rows, k_sel), lambda rt, c: (rt, 0)),
            pl.BlockSpec((r_rows, 1), lambda rt, c: (rt, 0)),
        ],
        out_specs=[
            pl.BlockSpec((r_rows, k_sel), lambda rt, c: (rt, 0)),
            pl.BlockSpec((r_rows, k_sel), lambda rt, c: (rt, 0)),
        ],
        out_shape=[
            jax.ShapeDtypeStruct((l_seq, k_sel), jnp.int32),
            jax.ShapeDtypeStruct((l_seq, k_sel), jnp.int32),
        ],
        scratch_shapes=[
            pltpu.VMEM((CS // 128, r_rows, 128), jnp.float32),
            pltpu.VMEM((CS // 128, r_rows, 128), jnp.int32),
        ],
        compiler_params=pltpu.CompilerParams(
            dimension_semantics=("arbitrary", "arbitrary"),
            vmem_limit_bytes=100 * 1024 * 1024,
        ),
    )(q, e_pad, sq, sel, rr, tid2d)


def kernel(POI_id, time_id, loc_embeds, k, user):
    key = jax.random.key(42)
    ks, kt = jax.random.split(key)
    sel = jax.random.randint(ks, (L_SEQ, K_NEG), 0, NUM_NEAREST,
                             dtype=jnp.int32)
    rr = jax.random.randint(kt, (L_SEQ, K_NEG), 0, NUM_TIMES - 1,
                            dtype=jnp.int32)

    v = loc_embeds.shape[0]
    nc = -(-v // CS)
    e_pad = jnp.pad(loc_embeds, ((0, nc * CS - v), (0, 0)))
    q = _gather_q(loc_embeds, POI_id)
    sq = _sq_prepass(loc_embeds, nc, v)
    tid2d = time_id.reshape(-1, 1).astype(jnp.int32)
    neg, tneg = _main(q, e_pad, sq, sel, rr, tid2d, nc, R_ROWS)
    probs = jnp.ones((L_SEQ, K_NEG), jnp.float32)
    return (neg, probs, tneg, jnp.ones((L_SEQ, K_NEG), jnp.float32))
